# baseline (speedup 1.0000x reference)
import functools

import jax
import jax.numpy as jnp
from jax import lax
from jax.experimental import pallas as pl
from jax.experimental.pallas import tpu as pltpu
from jax.experimental.pallas import tpu_sc as plsc

B = 4096
L = 200
LPAD = 224
HALF = LPAD // 2
E = 64
EP = 128
OUT = 10
V = 1000000
VB2 = 8192
SPLIT = 61 * VB2
NROWS = V - SPLIT
SENT = NROWS
NROWS_PAD = NROWS + 8

NC = 2
NS = 16
NW = NC * NS
ROWS_PER_W = B // NW

NB = 4
PH = 4
RPP = ROWS_PER_W // PH


def _pool_body(x2_hbm, cnt_hbm, table_hbm, out_hbm, idx_v, rows_v, p_buf,
               cnt_s, *sems):
    wid = lax.axis_index("s") * NC + lax.axis_index("c")
    base = wid * ROWS_PER_W

    pltpu.sync_copy(cnt_hbm.at[pl.ds(base, ROWS_PER_W)], cnt_s)

    neg = jnp.full((16,), -jnp.inf, dtype=jnp.float32)

    def issue(slot, i):
        for h in range(2):
            pltpu.async_copy(
                table_hbm.at[idx_v.at[2 * i + h]],
                rows_v.at[pl.ds((2 * slot + h) * HALF, HALF)], sems[slot])

    def drain(slot):
        for h in range(2):
            pltpu.make_async_copy(
                table_hbm.at[idx_v.at[h]],
                rows_v.at[pl.ds((2 * slot + h) * HALF, HALF)],
                sems[slot]).wait()

    for ph in range(PH):
        pltpu.sync_copy(
            x2_hbm.at[pl.ds(base * 2 + ph * 2 * RPP, 2 * RPP)], idx_v)

        for s in range(NB):
            issue(s, s)

        def blk_body(g, carry):
            for s in range(NB):
                i = g * NB + s
                c8v = cnt_s[ph * RPP + i, pl.ds(0, 16)]

                drain(s)

                def seq_body(j, acc):
                    lo_blk = jnp.full((16,), 8, jnp.int32) * j < c8v
                    accs = list(acc)
                    for jj in range(8):
                        r = rows_v.at[2 * s * HALF + j * 8 + jj]
                        for v in range(4):
                            val = jnp.where(lo_blk,
                                            r[pl.ds(v * 16, 16)],
                                            r[pl.ds(E + v * 16, 16)])
                            accs[v] = jnp.maximum(accs[v], val)
                    return tuple(accs)

                acc = lax.fori_loop(0, LPAD // 8, seq_body,
                                    (neg, neg, neg, neg))
                for v in range(4):
                    p_buf[i, pl.ds(v * 16, 16)] = acc[v]

                nxt = i + NB

                @pl.when(nxt < RPP)
                def _():
                    issue(s, nxt)
            return carry

        lax.fori_loop(0, RPP // NB, blk_body, 0)
        pltpu.sync_copy(p_buf, out_hbm.at[pl.ds(base + ph * RPP, RPP)])


_pool = functools.partial(
    pl.kernel,
    mesh=plsc.VectorSubcoreMesh(
        core_axis_name="c", subcore_axis_name="s",
        num_cores=NC, num_subcores=NS,
    ),
    out_type=jax.ShapeDtypeStruct((B, E), jnp.float32),
    scratch_types=[
        pltpu.VMEM((2 * RPP, HALF), jnp.int32),
        pltpu.VMEM((2 * NB * HALF, EP), jnp.float32),
        pltpu.VMEM((RPP, E), jnp.float32),
        pltpu.VMEM((ROWS_PER_W, 16), jnp.int32),
    ] + [pltpu.SemaphoreType.DMA] * NB,
    compiler_params=pltpu.CompilerParams(use_tc_tiling_on_sc=False),
)(_pool_body)


def _repack_body(lo_ref, hi_ref, o_ref):
    k = pl.program_id(0)
    lo_t = lo_ref[...].T
    hi_t = hi_ref[...].T
    row_g = k * VB2 + lax.broadcasted_iota(jnp.int32, (VB2, E), 0)
    m = row_g < SENT
    ninf = jnp.full((VB2, E), -jnp.inf, jnp.float32)
    o_ref[...] = jnp.concatenate(
        [jnp.where(m, lo_t, ninf), jnp.where(m, hi_t, ninf)], axis=1)


def _repack(table_t):
    grid = (NROWS_PAD + VB2 - 1) // VB2
    return pl.pallas_call(
        _repack_body,
        grid=(grid,),
        in_specs=[
            pl.BlockSpec((E, VB2), lambda k: (0, k)),
            pl.BlockSpec((E, VB2), lambda k: (0, 61 + k)),
        ],
        out_specs=pl.BlockSpec((VB2, EP), lambda k: (k, 0)),
        out_shape=jax.ShapeDtypeStruct((NROWS_PAD, EP), jnp.float32),
    )(table_t, table_t)


def _linear_body(p_ref, w_ref, b_ref, o_ref):
    h = jnp.maximum(p_ref[...], 0.0)
    o_ref[...] = (
        jnp.dot(h, w_ref[...], preferred_element_type=jnp.float32) + b_ref[...]
    )


def kernel(x, emb_table, fc_w, fc_b):
    x = x.astype(jnp.int32)
    x_pad = jnp.concatenate([x, x[:, L - 8:]], axis=1)
    hi = x_pad >= SPLIT
    phys = jnp.where(hi, x_pad - SPLIT, x_pad)
    hin = hi.astype(jnp.int32)
    rank_hi = jnp.cumsum(hin, axis=1) - hin
    rank_lo = jnp.arange(208, dtype=jnp.int32)[None, :] - rank_hi - hin
    n_lo = 208 - hin.sum(axis=1)
    c8 = ((n_lo + 7) // 8) * 8
    dst = jnp.where(hi, c8[:, None] + rank_hi, rank_lo)
    slots = jnp.full((B, LPAD), SENT, jnp.int32)
    slots = slots.at[jnp.arange(B)[:, None], dst].set(phys)
    x2 = slots.reshape(2 * B, HALF)

    table_pk = _repack(emb_table.T)
    cnt_rep = jnp.broadcast_to(c8[:, None], (B, 16))
    p = _pool(x2, cnt_rep, table_pk)

    out = pl.pallas_call(
        _linear_body,
        out_shape=jax.ShapeDtypeStruct((B, OUT), jnp.float32),
    )(p, fc_w.T, fc_b.reshape(1, OUT))
    return out


# speedup vs baseline: 13.8124x; 13.8124x over previous
import functools

import jax
import jax.numpy as jnp
from jax import lax
from jax.experimental import pallas as pl
from jax.experimental.pallas import tpu as pltpu
from jax.experimental.pallas import tpu_sc as plsc

B = 4096
L = 200
LPAD = 208
HALF = LPAD // 2
E = 64
EP = 128
OUT = 10

NC = 2
NS = 16
NW = NC * NS
ROWS_PER_W = B // NW

NB = 4
PH = 2
RPP = ROWS_PER_W // PH


def _pool_body(x2_hbm, table_hbm, out_hbm, idx_v, rows_v, p_buf, *sems):
    wid = lax.axis_index("s") * NC + lax.axis_index("c")
    base = wid * ROWS_PER_W

    neg = jnp.full((16,), -jnp.inf, dtype=jnp.float32)

    def issue(slot, i):
        for h in range(2):
            pltpu.async_copy(
                table_hbm.at[idx_v.at[2 * i + h]],
                rows_v.at[pl.ds((2 * slot + h) * HALF, HALF)], sems[slot])

    def drain(slot):
        for h in range(2):
            pltpu.make_async_copy(
                table_hbm.at[idx_v.at[h]],
                rows_v.at[pl.ds((2 * slot + h) * HALF, HALF)],
                sems[slot]).wait()

    for ph in range(PH):
        pltpu.sync_copy(
            x2_hbm.at[pl.ds(base * 2 + ph * 2 * RPP, 2 * RPP)], idx_v)

        for s in range(NB):
            issue(s, s)

        def blk_body(g, carry):
            for s in range(NB):
                i = g * NB + s
                drain(s)

                def seq_body(j, acc):
                    accs = list(acc)
                    for jj in range(8):
                        r = rows_v.at[2 * s * HALF + j * 8 + jj]
                        for v in range(4):
                            accs[v] = jnp.maximum(
                                accs[v], r[pl.ds(v * 16, 16)])
                    return tuple(accs)

                acc = lax.fori_loop(0, 2 * HALF // 8, seq_body,
                                    (neg, neg, neg, neg))
                for v in range(4):
                    p_buf[ph * RPP + i, pl.ds(v * 16, 16)] = acc[v]

                nxt = i + NB

                @pl.when(nxt < RPP)
                def _():
                    issue(s, nxt)
            return carry

        lax.fori_loop(0, RPP // NB, blk_body, 0)

    pltpu.sync_copy(p_buf, out_hbm.at[pl.ds(base, ROWS_PER_W)])


_pool = functools.partial(
    pl.kernel,
    mesh=plsc.VectorSubcoreMesh(
        core_axis_name="c", subcore_axis_name="s",
        num_cores=NC, num_subcores=NS,
    ),
    out_type=jax.ShapeDtypeStruct((B, E), jnp.float32),
    scratch_types=[
        pltpu.VMEM((2 * RPP, HALF), jnp.int32),
        pltpu.VMEM((2 * NB * HALF, EP), jnp.float32),
        pltpu.VMEM((ROWS_PER_W, E), jnp.float32),
    ] + [pltpu.SemaphoreType.DMA] * NB,
    compiler_params=pltpu.CompilerParams(use_tc_tiling_on_sc=False),
)(_pool_body)


VB = 32768


def _repack_body(t_ref, o_ref):
    bt = t_ref[...].T
    o_ref[...] = jnp.concatenate(
        [bt, jnp.zeros((VB, EP - E), jnp.float32)], axis=1)


def _repack(table_t):
    grid = (1000000 + VB - 1) // VB
    return pl.pallas_call(
        _repack_body,
        grid=(grid,),
        in_specs=[pl.BlockSpec((E, VB), lambda k: (0, k))],
        out_specs=pl.BlockSpec((VB, EP), lambda k: (k, 0)),
        out_shape=jax.ShapeDtypeStruct((1000000, EP), jnp.float32),
    )(table_t)


def _linear_body(p_ref, w_ref, b_ref, o_ref):
    h = jnp.maximum(p_ref[...], 0.0)
    o_ref[...] = (
        jnp.dot(h, w_ref[...], preferred_element_type=jnp.float32) + b_ref[...]
    )


def kernel(x, emb_table, fc_w, fc_b):
    x = x.astype(jnp.int32)
    x_pad = jnp.concatenate([x, x[:, L - (LPAD - L):]], axis=1)
    x2 = x_pad.reshape(2 * B, HALF)

    table_pad = _repack(emb_table.T)
    p = _pool(x2, table_pad)

    out = pl.pallas_call(
        _linear_body,
        out_shape=jax.ShapeDtypeStruct((B, OUT), jnp.float32),
    )(p, fc_w.T, fc_b.reshape(1, OUT))
    return out
